# baseline (speedup 1.0000x reference)
import functools

import jax
import jax.numpy as jnp
from jax import lax
from jax.experimental import pallas as pl
from jax.experimental.pallas import tpu as pltpu
from jax.experimental.pallas import tpu_sc as plsc

NC = 2
NS = 16
NW = NC * NS
CHUNK = 128


@functools.partial(jax.jit, static_argnames=("total", "embed_dim", "j_per_w"))
def _fused_gather(cat3, off3, weight, *, total, embed_dim, j_per_w):
    b_per_w = j_per_w * CHUNK
    mesh = plsc.VectorSubcoreMesh(core_axis_name="c", subcore_axis_name="s")

    @functools.partial(
        pl.kernel,
        out_type=jax.ShapeDtypeStruct((total, embed_dim), jnp.float32),
        mesh=mesh,
        compiler_params=pltpu.CompilerParams(use_tc_tiling_on_sc=False),
        scratch_types=[
            pltpu.VMEM((j_per_w, CHUNK), jnp.int32),
            pltpu.VMEM((j_per_w, CHUNK), jnp.int32),
            pltpu.VMEM((j_per_w, CHUNK), jnp.int32),
            pltpu.VMEM((CHUNK, embed_dim), jnp.float32),
            pltpu.SemaphoreType.DMA,
        ],
    )
    def run(cat_hbm, off_hbm, w_hbm, out_hbm, cat_v, off_v, idx_v, rows_v, sem):
        wid = lax.axis_index("s") * NC + lax.axis_index("c")
        pltpu.sync_copy(cat_hbm.at[wid], cat_v)
        pltpu.sync_copy(off_hbm.at[wid], off_v)

        def add_body(t, carry):
            j = t // (CHUNK // 16)
            i = (t % (CHUNK // 16)) * 16
            idx_v[j, pl.ds(i, 16)] = cat_v[j, pl.ds(i, 16)] + off_v[j, pl.ds(i, 16)]
            return carry

        lax.fori_loop(0, j_per_w * (CHUNK // 16), add_body, 0)

        base = wid * b_per_w

        def gather_body(j, carry):
            pltpu.async_copy(w_hbm.at[idx_v.at[j]], rows_v, sem).wait()
            pltpu.sync_copy(rows_v, out_hbm.at[pl.ds(base + j * CHUNK, CHUNK)])
            return carry

        lax.fori_loop(0, j_per_w, gather_body, 0)

    return run(cat3, off3, weight)


def kernel(categorical_inputs, weight):
    B, F = categorical_inputs.shape
    R, D = weight.shape
    total = B * F
    assert total % (NW * CHUNK) == 0
    j_per_w = total // (NW * CHUNK)

    offsets = (jnp.arange(F, dtype=jnp.int32) * (R // F))[None, :]
    cat3 = categorical_inputs.reshape(NW, j_per_w, CHUNK)
    off3 = jnp.broadcast_to(offsets, (B, F)).reshape(NW, j_per_w, CHUNK)
    out = _fused_gather(cat3, off3, weight,
                        total=total, embed_dim=D, j_per_w=j_per_w)
    return out.reshape(B, F, D)


# speedup vs baseline: 1.0415x; 1.0415x over previous
import functools

import jax
import jax.numpy as jnp
from jax import lax
from jax.experimental import pallas as pl
from jax.experimental.pallas import tpu as pltpu
from jax.experimental.pallas import tpu_sc as plsc

NC = 2
NS = 16
NW = NC * NS
CHUNK = 128
KBUF = 4


@functools.partial(jax.jit, static_argnames=("total", "embed_dim", "j_per_w", "num_fields"))
def _fused_gather(cat3, weight, *, total, embed_dim, j_per_w, num_fields):
    b_per_w = j_per_w * CHUNK
    per_table = weight.shape[0] // num_fields
    mesh = plsc.VectorSubcoreMesh(core_axis_name="c", subcore_axis_name="s")
    n_groups = j_per_w // KBUF

    @functools.partial(
        pl.kernel,
        out_type=jax.ShapeDtypeStruct((total, embed_dim), jnp.float32),
        mesh=mesh,
        compiler_params=pltpu.CompilerParams(use_tc_tiling_on_sc=False),
        scratch_types=[
            pltpu.VMEM((j_per_w, CHUNK), jnp.int32),
            pltpu.VMEM((2, KBUF, CHUNK, embed_dim), jnp.float32),
            pltpu.SemaphoreType.DMA((2,)),
            pltpu.SemaphoreType.DMA((2,)),
        ],
    )
    def run(cat_hbm, w_hbm, out_hbm, idx_v, rows_v, gsem, wsem):
        wid = lax.axis_index("s") * NC + lax.axis_index("c")
        pltpu.sync_copy(cat_hbm.at[wid], idx_v)

        lane = lax.iota(jnp.int32, 16)

        def add_body(t, carry):
            j = t // (CHUNK // 16)
            i = (t % (CHUNK // 16)) * 16
            field = (t * 16 + lane) % num_fields
            idx_v[j, pl.ds(i, 16)] = idx_v[j, pl.ds(i, 16)] + field * per_table
            return carry

        lax.fori_loop(0, j_per_w * (CHUNK // 16), add_body, 0)

        base = wid * b_per_w

        def start_gathers(g):
            bank = g % 2
            for b in range(KBUF):
                j = g * KBUF + b
                pltpu.async_copy(w_hbm.at[idx_v.at[j]], rows_v.at[bank, b],
                                 gsem.at[bank])

        def drain(descs):
            for d in descs:
                d.wait()

        def gather_waits(g):
            bank = g % 2
            return [pltpu.make_async_copy(w_hbm.at[idx_v.at[g * KBUF + b]],
                                          rows_v.at[bank, b], gsem.at[bank])
                    for b in range(KBUF)]

        def start_writes(g):
            bank = g % 2
            descs = []
            for b in range(KBUF):
                j = g * KBUF + b
                descs.append(pltpu.async_copy(
                    rows_v.at[bank, b],
                    out_hbm.at[pl.ds(base + j * CHUNK, CHUNK)],
                    wsem.at[bank]))
            return descs

        start_gathers(0)
        wr = {}
        for g in range(n_groups):
            if g >= 1:
                drain(wr.pop(g - 1))
            if g + 1 < n_groups:
                start_gathers(g + 1)
            drain(gather_waits(g))
            wr[g] = start_writes(g)
        drain(wr.pop(n_groups - 1))

    return run(cat3, weight)


def kernel(categorical_inputs, weight):
    B, F = categorical_inputs.shape
    R, D = weight.shape
    total = B * F
    assert total % (NW * CHUNK) == 0
    j_per_w = total // (NW * CHUNK)
    assert j_per_w % KBUF == 0

    cat3 = categorical_inputs.reshape(NW, j_per_w, CHUNK)
    out = _fused_gather(cat3, weight, total=total, embed_dim=D,
                        j_per_w=j_per_w, num_fields=F)
    return out.reshape(B, F, D)
